# baseline (speedup 1.0000x reference)
import functools
import random

import jax
import jax.numpy as jnp
import numpy as np
from jax.experimental import pallas as pl
from jax.experimental.pallas import tpu as pltpu

W = 690
SMAX = W + 2
T_LEN = SMAX + 2
ROWS = 16 * 1024
ROW_BLOCK = 512
PER = float(ROWS)
F32 = jnp.float32


def _build_perm_table() -> np.ndarray:
    rows = []
    for m in range(SMAX):
        order = list(range(m))
        rng = random.Random(42)
        rng.shuffle(order)
        rows.append(order + list(range(m, SMAX)))
    return np.array(rows, dtype=np.float32)


_PERM_TABLE = _build_perm_table()


def _colsum_kernel(x_ref, out_ref):
    i = pl.program_id(0)

    @pl.when(i == 0)
    def _():
        out_ref[...] = jnp.zeros_like(out_ref)

    out_ref[...] += jnp.sum(x_ref[...], axis=0, keepdims=True)


def _iota(shape, dim):
    return jax.lax.broadcasted_iota(jnp.int32, shape, dim).astype(F32)


def _perm_kernel(colsum_ref, ptab_ref, p_ref):
    f = F32
    colsum = colsum_ref[...]
    ar = _iota((1, W), 1)

    p = jnp.sum(colsum) / (PER * W)
    iota_c = _iota((W, W), 0)
    iota_i = _iota((W, W), 1)
    band = ((iota_c >= iota_i) & (iota_c <= iota_i + 3)).astype(f)
    win = jnp.dot(colsum, band, preferred_element_type=f,
                  precision=jax.lax.Precision.HIGHEST)
    w = jnp.minimum(4.0, jnp.float32(W) - ar)
    mean_value = win / (PER * w)
    maskf = (mean_value >= p).astype(f)

    zero1 = jnp.zeros((1, 1), f)
    prev = jnp.concatenate([zero1, maskf[:, :-1]], axis=1)
    nxt = jnp.concatenate([maskf[:, 1:], zero1], axis=1)
    run_start = maskf * (1.0 - prev)
    run_end = maskf * (1.0 - nxt)
    G = jnp.sum(run_start)

    ltw = (iota_c <= iota_i).astype(f)
    cs_start = jnp.dot(run_start, ltw, preferred_element_type=f,
                  precision=jax.lax.Precision.HIGHEST)
    cs_end = jnp.dot(run_end, ltw, preferred_element_type=f,
                  precision=jax.lax.Precision.HIGHEST)
    iota_wk_w = _iota((W, SMAX), 0)
    iota_wk_k = _iota((W, SMAX), 1)
    o_s = (run_start.reshape(W, 1) * (cs_start.reshape(W, 1) - 1.0 == iota_wk_k))
    o_e = (run_end.reshape(W, 1) * (cs_end.reshape(W, 1) - 1.0 == iota_wk_k))
    del iota_wk_w
    k_ar = _iota((1, SMAX), 1)
    tail = jnp.float32(W) * (k_ar >= G).astype(f)
    firsts = jnp.dot(ar, o_s, preferred_element_type=f,
                  precision=jax.lax.Precision.HIGHEST) + tail
    lasts = jnp.dot(ar, o_e, preferred_element_type=f,
                  precision=jax.lax.Precision.HIGHEST) + tail

    iota_kt_k = _iota((SMAX, T_LEN), 0)
    iota_kt_t = _iota((SMAX, T_LEN), 1)
    ef = (iota_kt_t == 2.0 * iota_kt_k).astype(f)
    el = (iota_kt_t == 2.0 * iota_kt_k + 1.0).astype(f)
    part = (jnp.dot(firsts, ef, preferred_element_type=f,
                  precision=jax.lax.Precision.HIGHEST)
            + jnp.dot(lasts, el, preferred_element_type=f,
                  precision=jax.lax.Precision.HIGHEST))
    t_ar = _iota((1, T_LEN), 1)
    full = jnp.where(t_ar < 2.0 * G, part, jnp.float32(W))
    cond0 = (firsts[:, :1] == 0.0)
    full_sh = jnp.concatenate([zero1, full[:, :-1]], axis=1)
    bp = jnp.where(cond0, full, full_sh)
    n = 2.0 * G + 1.0 - jnp.sum(cond0.astype(f))
    starts = bp[:, :SMAX]
    ends_b = bp[:, 1:SMAX + 1]

    onehot_n = (k_ar == n).astype(f)
    perm = jnp.dot(onehot_n, ptab_ref[...], preferred_element_type=f,
                  precision=jax.lax.Precision.HIGHEST)
    iota_kk_a = _iota((SMAX, SMAX), 0)
    iota_kk_b = _iota((SMAX, SMAX), 1)
    p1 = (iota_kk_a == perm.reshape(1, SMAX)).astype(f)
    s_sh = jnp.dot(starts, p1, preferred_element_type=f,
                  precision=jax.lax.Precision.HIGHEST)
    e_sh = jnp.dot(ends_b, p1, preferred_element_type=f,
                  precision=jax.lax.Precision.HIGHEST)

    seg_l = jnp.where(k_ar < n, e_sh - s_sh, 0.0)
    lts = (iota_kk_a <= iota_kk_b).astype(f)
    ends_c = jnp.dot(seg_l, lts, preferred_element_type=f,
                  precision=jax.lax.Precision.HIGHEST)
    iota_kw_k = _iota((SMAX, W), 0)
    iota_kw_p = _iota((SMAX, W), 1)
    sid = jnp.sum((ends_c.reshape(SMAX, 1) <= iota_kw_p).astype(f),
                  axis=0, keepdims=True)
    bk = s_sh - ends_c + seg_l
    o2 = (sid.reshape(1, W) == iota_kw_k).astype(f)
    col = jnp.dot(bk, o2, preferred_element_type=f,
                  precision=jax.lax.Precision.HIGHEST) + ar

    p_ref[...] = (iota_c == col.reshape(1, W)).astype(f)


def _permute_kernel(x_ref, p_ref, out_ref):
    out_ref[...] = jnp.dot(x_ref[...], p_ref[...], preferred_element_type=F32,
                           precision=jax.lax.Precision.HIGHEST)


@jax.jit
def kernel(img):
    x = img.reshape(ROWS, W)
    n_blocks = ROWS // ROW_BLOCK

    colsum = pl.pallas_call(
        _colsum_kernel,
        grid=(n_blocks,),
        in_specs=[pl.BlockSpec((ROW_BLOCK, W), lambda i: (i, 0))],
        out_specs=pl.BlockSpec((1, W), lambda i: (0, 0)),
        out_shape=jax.ShapeDtypeStruct((1, W), F32),
    )(x)

    ptab = jnp.asarray(_PERM_TABLE)
    p_mat = pl.pallas_call(
        _perm_kernel,
        out_shape=jax.ShapeDtypeStruct((W, W), F32),
    )(colsum, ptab)

    out = pl.pallas_call(
        _permute_kernel,
        grid=(n_blocks,),
        in_specs=[
            pl.BlockSpec((ROW_BLOCK, W), lambda i: (i, 0)),
            pl.BlockSpec((W, W), lambda i: (0, 0)),
        ],
        out_specs=pl.BlockSpec((ROW_BLOCK, W), lambda i: (i, 0)),
        out_shape=jax.ShapeDtypeStruct((ROWS, W), F32),
    )(x, p_mat)

    return out.reshape(img.shape)


# speedup vs baseline: 1.3807x; 1.3807x over previous
import functools
import random

import jax
import jax.numpy as jnp
import numpy as np
from jax.experimental import pallas as pl
from jax.experimental.pallas import tpu as pltpu

W = 690
SMAX = W + 2
T_LEN = SMAX + 2
ROWS = 16 * 1024
ROW_BLOCK = 512
PER = float(ROWS)
F32 = jnp.float32


def _build_perm_table() -> np.ndarray:
    rows = []
    for m in range(SMAX):
        order = list(range(m))
        rng = random.Random(42)
        rng.shuffle(order)
        rows.append(order + list(range(m, SMAX)))
    return np.array(rows, dtype=np.float32)


_PERM_TABLE = _build_perm_table()


def _colsum_kernel(x_ref, out_ref):
    b = pl.program_id(0)
    h = pl.program_id(1)

    @pl.when((b == 0) & (h == 0))
    def _():
        out_ref[...] = jnp.zeros_like(out_ref)

    out_ref[0] += jnp.sum(x_ref[0], axis=0, keepdims=True)


def _iota(shape, dim):
    return jax.lax.broadcasted_iota(jnp.int32, shape, dim).astype(F32)


def _perm_kernel(colsum_ref, ptab_ref, p_ref):
    f = F32
    colsum = colsum_ref[0]
    ar = _iota((1, W), 1)

    p = jnp.sum(colsum) / (PER * W)
    iota_c = _iota((W, W), 0)
    iota_i = _iota((W, W), 1)
    band = ((iota_c >= iota_i) & (iota_c <= iota_i + 3)).astype(f)
    win = jnp.dot(colsum, band, preferred_element_type=f,
                  precision=jax.lax.Precision.HIGHEST)
    w = jnp.minimum(4.0, jnp.float32(W) - ar)
    mean_value = win / (PER * w)
    maskf = (mean_value >= p).astype(f)

    zero1 = jnp.zeros((1, 1), f)
    prev = jnp.concatenate([zero1, maskf[:, :-1]], axis=1)
    nxt = jnp.concatenate([maskf[:, 1:], zero1], axis=1)
    run_start = maskf * (1.0 - prev)
    run_end = maskf * (1.0 - nxt)
    G = jnp.sum(run_start)

    ltw = (iota_c <= iota_i).astype(f)
    cs_start = jnp.dot(run_start, ltw, preferred_element_type=f,
                  precision=jax.lax.Precision.HIGHEST)
    cs_end = jnp.dot(run_end, ltw, preferred_element_type=f,
                  precision=jax.lax.Precision.HIGHEST)
    iota_wk_w = _iota((W, SMAX), 0)
    iota_wk_k = _iota((W, SMAX), 1)
    o_s = (run_start.reshape(W, 1) * (cs_start.reshape(W, 1) - 1.0 == iota_wk_k))
    o_e = (run_end.reshape(W, 1) * (cs_end.reshape(W, 1) - 1.0 == iota_wk_k))
    del iota_wk_w
    k_ar = _iota((1, SMAX), 1)
    tail = jnp.float32(W) * (k_ar >= G).astype(f)
    firsts = jnp.dot(ar, o_s, preferred_element_type=f,
                  precision=jax.lax.Precision.HIGHEST) + tail
    lasts = jnp.dot(ar, o_e, preferred_element_type=f,
                  precision=jax.lax.Precision.HIGHEST) + tail

    iota_kt_k = _iota((SMAX, T_LEN), 0)
    iota_kt_t = _iota((SMAX, T_LEN), 1)
    ef = (iota_kt_t == 2.0 * iota_kt_k).astype(f)
    el = (iota_kt_t == 2.0 * iota_kt_k + 1.0).astype(f)
    part = (jnp.dot(firsts, ef, preferred_element_type=f,
                  precision=jax.lax.Precision.HIGHEST)
            + jnp.dot(lasts, el, preferred_element_type=f,
                  precision=jax.lax.Precision.HIGHEST))
    t_ar = _iota((1, T_LEN), 1)
    full = jnp.where(t_ar < 2.0 * G, part, jnp.float32(W))
    cond0 = (firsts[:, :1] == 0.0)
    full_sh = jnp.concatenate([zero1, full[:, :-1]], axis=1)
    bp = jnp.where(cond0, full, full_sh)
    n = 2.0 * G + 1.0 - jnp.sum(cond0.astype(f))
    starts = bp[:, :SMAX]
    ends_b = bp[:, 1:SMAX + 1]

    onehot_n = (k_ar == n).astype(f)
    perm = jnp.dot(onehot_n, ptab_ref[...], preferred_element_type=f,
                  precision=jax.lax.Precision.HIGHEST)
    iota_kk_a = _iota((SMAX, SMAX), 0)
    iota_kk_b = _iota((SMAX, SMAX), 1)
    p1 = (iota_kk_a == perm.reshape(1, SMAX)).astype(f)
    s_sh = jnp.dot(starts, p1, preferred_element_type=f,
                  precision=jax.lax.Precision.HIGHEST)
    e_sh = jnp.dot(ends_b, p1, preferred_element_type=f,
                  precision=jax.lax.Precision.HIGHEST)

    seg_l = jnp.where(k_ar < n, e_sh - s_sh, 0.0)
    lts = (iota_kk_a <= iota_kk_b).astype(f)
    ends_c = jnp.dot(seg_l, lts, preferred_element_type=f,
                  precision=jax.lax.Precision.HIGHEST)
    iota_kw_k = _iota((SMAX, W), 0)
    iota_kw_p = _iota((SMAX, W), 1)
    sid = jnp.sum((ends_c.reshape(SMAX, 1) <= iota_kw_p).astype(f),
                  axis=0, keepdims=True)
    bk = s_sh - ends_c + seg_l
    o2 = (sid.reshape(1, W) == iota_kw_k).astype(f)
    col = jnp.dot(bk, o2, preferred_element_type=f,
                  precision=jax.lax.Precision.HIGHEST) + ar

    p_ref[...] = (iota_c == col.reshape(1, W)).astype(f)


def _permute_kernel(x_ref, p_ref, out_ref):
    x = x_ref[0]
    p = p_ref[...].astype(jnp.bfloat16)
    x_hi = x.astype(jnp.bfloat16)
    x_lo = (x - x_hi.astype(F32)).astype(jnp.bfloat16)
    out_ref[0] = (jnp.dot(x_hi, p, preferred_element_type=F32)
                  + jnp.dot(x_lo, p, preferred_element_type=F32))


@jax.jit
def kernel(img):
    nb, nh = img.shape[0], img.shape[1] // ROW_BLOCK

    colsum = pl.pallas_call(
        _colsum_kernel,
        grid=(nb, nh),
        in_specs=[pl.BlockSpec((1, ROW_BLOCK, W), lambda b, h: (b, h, 0))],
        out_specs=pl.BlockSpec((1, 1, W), lambda b, h: (0, 0, 0)),
        out_shape=jax.ShapeDtypeStruct((1, 1, W), F32),
    )(img)

    ptab = jnp.asarray(_PERM_TABLE)
    p_mat = pl.pallas_call(
        _perm_kernel,
        out_shape=jax.ShapeDtypeStruct((W, W), F32),
    )(colsum, ptab)

    out = pl.pallas_call(
        _permute_kernel,
        grid=(nb, nh),
        in_specs=[
            pl.BlockSpec((1, ROW_BLOCK, W), lambda b, h: (b, h, 0)),
            pl.BlockSpec((W, W), lambda b, h: (0, 0)),
        ],
        out_specs=pl.BlockSpec((1, ROW_BLOCK, W), lambda b, h: (b, h, 0)),
        out_shape=jax.ShapeDtypeStruct(img.shape, F32),
    )(img, p_mat)

    return out


# speedup vs baseline: 1.4619x; 1.0589x over previous
import functools
import random

import jax
import jax.numpy as jnp
import numpy as np
from jax.experimental import pallas as pl
from jax.experimental.pallas import tpu as pltpu

W = 690
SMAX = W + 2
T_LEN = SMAX + 2
ROWS = 16 * 1024
ROW_BLOCK = 512
PER = float(ROWS)
F32 = jnp.float32


def _build_perm_table() -> np.ndarray:
    rows = []
    for m in range(SMAX):
        order = list(range(m))
        rng = random.Random(42)
        rng.shuffle(order)
        rows.append(order + list(range(m, SMAX)))
    return np.array(rows, dtype=np.float32)


_PERM_TABLE = _build_perm_table()


def _colsum_kernel(x_ref, out_ref):
    out_ref[...] = jnp.sum(x_ref[...], axis=(0, 1), keepdims=False)[None, None, :]


def _iota(shape, dim):
    return jax.lax.broadcasted_iota(jnp.int32, shape, dim).astype(F32)


def _perm_kernel(colsum_ref, ptab_ref, p_ref):
    f = F32
    colsum = jnp.sum(colsum_ref[...], axis=(0, 1), keepdims=False)[None, :]
    ar = _iota((1, W), 1)

    p = jnp.sum(colsum) / (PER * W)
    iota_c = _iota((W, W), 0)
    iota_i = _iota((W, W), 1)
    band = ((iota_c >= iota_i) & (iota_c <= iota_i + 3)).astype(f)
    win = jnp.dot(colsum, band, preferred_element_type=f,
                  precision=jax.lax.Precision.HIGHEST)
    w = jnp.minimum(4.0, jnp.float32(W) - ar)
    mean_value = win / (PER * w)
    maskf = (mean_value >= p).astype(f)

    zero1 = jnp.zeros((1, 1), f)
    prev = jnp.concatenate([zero1, maskf[:, :-1]], axis=1)
    nxt = jnp.concatenate([maskf[:, 1:], zero1], axis=1)
    run_start = maskf * (1.0 - prev)
    run_end = maskf * (1.0 - nxt)
    G = jnp.sum(run_start)

    ltw = (iota_c <= iota_i).astype(f)
    cs_start = jnp.dot(run_start, ltw, preferred_element_type=f,
                  precision=jax.lax.Precision.HIGHEST)
    cs_end = jnp.dot(run_end, ltw, preferred_element_type=f,
                  precision=jax.lax.Precision.HIGHEST)
    iota_wk_w = _iota((W, SMAX), 0)
    iota_wk_k = _iota((W, SMAX), 1)
    o_s = (run_start.reshape(W, 1) * (cs_start.reshape(W, 1) - 1.0 == iota_wk_k))
    o_e = (run_end.reshape(W, 1) * (cs_end.reshape(W, 1) - 1.0 == iota_wk_k))
    del iota_wk_w
    k_ar = _iota((1, SMAX), 1)
    tail = jnp.float32(W) * (k_ar >= G).astype(f)
    firsts = jnp.dot(ar, o_s, preferred_element_type=f,
                  precision=jax.lax.Precision.HIGHEST) + tail
    lasts = jnp.dot(ar, o_e, preferred_element_type=f,
                  precision=jax.lax.Precision.HIGHEST) + tail

    iota_kt_k = _iota((SMAX, T_LEN), 0)
    iota_kt_t = _iota((SMAX, T_LEN), 1)
    ef = (iota_kt_t == 2.0 * iota_kt_k).astype(f)
    el = (iota_kt_t == 2.0 * iota_kt_k + 1.0).astype(f)
    part = (jnp.dot(firsts, ef, preferred_element_type=f,
                  precision=jax.lax.Precision.HIGHEST)
            + jnp.dot(lasts, el, preferred_element_type=f,
                  precision=jax.lax.Precision.HIGHEST))
    t_ar = _iota((1, T_LEN), 1)
    full = jnp.where(t_ar < 2.0 * G, part, jnp.float32(W))
    cond0 = (firsts[:, :1] == 0.0)
    full_sh = jnp.concatenate([zero1, full[:, :-1]], axis=1)
    bp = jnp.where(cond0, full, full_sh)
    n = 2.0 * G + 1.0 - jnp.sum(cond0.astype(f))
    starts = bp[:, :SMAX]
    ends_b = bp[:, 1:SMAX + 1]

    onehot_n = (k_ar == n).astype(f)
    perm = jnp.dot(onehot_n, ptab_ref[...], preferred_element_type=f,
                  precision=jax.lax.Precision.HIGHEST)
    iota_kk_a = _iota((SMAX, SMAX), 0)
    iota_kk_b = _iota((SMAX, SMAX), 1)
    p1 = (iota_kk_a == perm.reshape(1, SMAX)).astype(f)
    s_sh = jnp.dot(starts, p1, preferred_element_type=f,
                  precision=jax.lax.Precision.HIGHEST)
    e_sh = jnp.dot(ends_b, p1, preferred_element_type=f,
                  precision=jax.lax.Precision.HIGHEST)

    seg_l = jnp.where(k_ar < n, e_sh - s_sh, 0.0)
    lts = (iota_kk_a <= iota_kk_b).astype(f)
    ends_c = jnp.dot(seg_l, lts, preferred_element_type=f,
                  precision=jax.lax.Precision.HIGHEST)
    iota_kw_k = _iota((SMAX, W), 0)
    iota_kw_p = _iota((SMAX, W), 1)
    sid = jnp.sum((ends_c.reshape(SMAX, 1) <= iota_kw_p).astype(f),
                  axis=0, keepdims=True)
    bk = s_sh - ends_c + seg_l
    o2 = (sid.reshape(1, W) == iota_kw_k).astype(f)
    col = jnp.dot(bk, o2, preferred_element_type=f,
                  precision=jax.lax.Precision.HIGHEST) + ar

    p_ref[...] = (iota_c == col.reshape(1, W)).astype(f)


def _permute_kernel(x_ref, p_ref, out_ref):
    x = x_ref[0]
    p = p_ref[...].astype(jnp.bfloat16)
    x_hi = x.astype(jnp.bfloat16)
    x_lo = (x - x_hi.astype(F32)).astype(jnp.bfloat16)
    out_ref[0] = (jnp.dot(x_hi, p, preferred_element_type=F32)
                  + jnp.dot(x_lo, p, preferred_element_type=F32))


@jax.jit
def kernel(img):
    nb, nh = img.shape[0], img.shape[1] // ROW_BLOCK

    ng = 8
    partial = pl.pallas_call(
        _colsum_kernel,
        grid=(ng,),
        in_specs=[pl.BlockSpec((img.shape[0] // ng, img.shape[1], W),
                               lambda i: (i, 0, 0))],
        out_specs=pl.BlockSpec((1, 1, W), lambda i: (i, 0, 0)),
        out_shape=jax.ShapeDtypeStruct((ng, 1, W), F32),
    )(img)

    ptab = jnp.asarray(_PERM_TABLE)
    p_mat = pl.pallas_call(
        _perm_kernel,
        out_shape=jax.ShapeDtypeStruct((W, W), F32),
    )(partial, ptab)

    out = pl.pallas_call(
        _permute_kernel,
        grid=(nb, nh),
        in_specs=[
            pl.BlockSpec((1, ROW_BLOCK, W), lambda b, h: (b, h, 0)),
            pl.BlockSpec((W, W), lambda b, h: (0, 0)),
        ],
        out_specs=pl.BlockSpec((1, ROW_BLOCK, W), lambda b, h: (b, h, 0)),
        out_shape=jax.ShapeDtypeStruct(img.shape, F32),
    )(img, p_mat)

    return out
